# baseline (speedup 1.0000x reference)
import functools

import jax
import jax.numpy as jnp
from jax import lax
from jax.experimental import pallas as pl
from jax.experimental.pallas import tpu as pltpu
from jax.experimental.pallas import tpu_sc as plsc

N = 10000
E = 320000
D = 128
HD = 64
AC = 144
NC = 2
NS = 16
NW = NC * NS
C = 32
NCHUNK = E // C
NITER = (NCHUNK + NW - 1) // NW
INIT_W = 10
ROWS_PT = N // INIT_W
RB = 2000

_TARR = tuple(32 * m + 2 * i + r
              for m in range(4) for r in range(2) for i in range(16))


def _proj_body(h_ref, ws_ref, wd_ref, wvt_ref, bd_ref, aw_ref,
               stab_ref, dtab_ref, init_ref):
    h = h_ref[...]
    s = jnp.dot(h, ws_ref[...], preferred_element_type=jnp.float32)
    dt = jnp.dot(h, wd_ref[...], preferred_element_type=jnp.float32) + bd_ref[0]
    stab_ref[...] = s.astype(jnp.bfloat16)
    dtab_ref[...] = dt.astype(jnp.bfloat16)
    t = s[:, :D] + dt
    zlr = 0.6 * t + 0.4 * jnp.abs(t)
    aw = aw_ref[0]
    l0 = jnp.sum(zlr[:, :HD] * aw[:HD], axis=1)
    l1 = jnp.sum(zlr[:, HD:] * aw[HD:], axis=1)
    w0 = jnp.exp(l0)
    w1 = jnp.exp(l1)
    vq = jnp.dot(h, wvt_ref[...], preferred_element_type=jnp.float32)
    msg = jnp.concatenate(
        [w0[:, None] * vq[:, :HD],
         w1[:, None] * vq[:, HD:],
         w0[:, None], w1[:, None],
         jnp.zeros((s.shape[0], AC - D - 2), jnp.float32)], axis=1)
    init_ref[...] = 0.5 * msg


def _final_body(acc_ref, ow_ref, ob_ref, pw_ref, pb_ref, out_ref):
    a = acc_ref[0] + acc_ref[1]
    hc = jnp.concatenate(
        [a[:, :HD] / a[:, D:D + 1], a[:, HD:D] / a[:, D + 1:D + 2]], axis=1)
    h2 = jnp.maximum(
        jnp.dot(hc, ow_ref[...], preferred_element_type=jnp.float32) + ob_ref[0],
        0.0)
    lg = jnp.dot(h2, pw_ref[...], preferred_element_type=jnp.float32) + pb_ref[0]
    m = jnp.max(lg, axis=1, keepdims=True)
    e = jnp.exp(lg - m)
    out_ref[...] = e / jnp.sum(e, axis=1, keepdims=True)


def _compute_chunk(srows_b, drows_b, msg_b, aw_regs, lane):
    @plsc.parallel_loop(0, C, unroll=4)
    def eloop(e):
        ls = []
        for h in range(2):
            parts = []
            for j in (2 * h, 2 * h + 1):
                t = (srows_b[e, pl.ds(j * 32, 32)] +
                     drows_b[e, pl.ds(j * 32, 32)])
                ta, tb = plsc.unpack(t, format=plsc.PackFormat.INTERLEAVED,
                                     preferred_element_type=jnp.float32)
                for r, tt in enumerate((ta, tb)):
                    lr = 0.6 * tt + 0.4 * jnp.abs(tt)
                    parts.append(lr * aw_regs[2 * j + r])
            ls.append(jnp.sum((parts[0] + parts[1]) + (parts[2] + parts[3])))
        w0 = jnp.exp(jnp.full((16,), ls[0], jnp.float32))
        w1 = jnp.exp(jnp.full((16,), ls[1], jnp.float32))
        msg_b[e, pl.ds(D, 16)] = jnp.where(
            lane == 0, w0, jnp.where(lane == 1, w1, 0.0))
        for m in range(4):
            w = w0 if m < 2 else w1
            v = srows_b[e, pl.ds(D + m * 32, 32)]
            va, vb = plsc.unpack(v, format=plsc.PackFormat.INTERLEAVED,
                                 preferred_element_type=jnp.float32)
            msg_b[e, pl.ds(m * 32, 16)] = va * w
            msg_b[e, pl.ds(m * 32 + 16, 16)] = vb * w


NB = 3


def _edge_body(ei_hbm, stab_hbm, dtab_hbm, aw_hbm, init_hbm, out_hbm,
               ibuf, scidx, srows, drows, msg, aw_v, shared,
               isem0, isem1, isem2, ssem0, ssem1, ssem2,
               dsem0, dsem1, dsem2, csem0, csem1, csem2):
    c = lax.axis_index("c")
    s = lax.axis_index("s")
    wid = s * NC + c

    isems = (isem0, isem1, isem2)
    ssems = (ssem0, ssem1, ssem2)
    dsems = (dsem0, dsem1, dsem2)
    csems = (csem0, csem1, csem2)

    pltpu.sync_copy(aw_hbm, aw_v)
    aw_regs = [aw_v[pl.ds(j * 16, 16)] for j in range(8)]
    lane = lax.iota(jnp.int32, 16)

    row0 = s * ROWS_PT

    @pl.when(s < INIT_W)
    def _init():
        pltpu.sync_copy(init_hbm.at[pl.ds(row0, ROWS_PT)],
                        shared.at[pl.ds(row0, ROWS_PT)])

    plsc.subcore_barrier()

    def _issue_idx(j, b):
        return pltpu.async_copy(
            ei_hbm.at[:, pl.ds((j * NW + wid) * C, C)], ibuf.at[b], isems[b])

    def _issue_gathers(b):
        pltpu.async_copy(stab_hbm.at[ibuf.at[b, 0]], srows.at[b], ssems[b])
        pltpu.async_copy(dtab_hbm.at[ibuf.at[b, 1]], drows.at[b], dsems[b])

    def _wait_gathers(b):
        pltpu.make_async_copy(stab_hbm.at[ibuf.at[b, 0]], srows.at[b],
                              ssems[b]).wait()
        pltpu.make_async_copy(dtab_hbm.at[ibuf.at[b, 1]], drows.at[b],
                              dsems[b]).wait()

    _issue_idx(0, 0).wait()
    d1 = _issue_idx(1, 1)
    _issue_gathers(0)
    d1.wait()
    _issue_gathers(1)
    _issue_idx(2, 2)

    def outer(i, _):
        for b in range(NB):
            jj = i * NB + b
            g2 = (b + 2) % NB

            def _valid(k):
                return ((jj + k) * NW + wid) < NCHUNK

            @pl.when(_valid(0))
            def _wg():
                _wait_gathers(b)

            @pl.when(_valid(2))
            def _nxt():
                pltpu.make_async_copy(
                    ei_hbm.at[:, pl.ds(((jj + 2) * NW + wid) * C, C)],
                    ibuf.at[g2], isems[g2]).wait()
                _issue_gathers(g2)

            @pl.when((i >= 1) & _valid(0))
            def _wsc():
                pltpu.make_async_copy(msg.at[b], shared.at[scidx.at[b]],
                                      csems[b]).wait()

            @pl.when(_valid(0))
            def _cmp():
                scidx[b, pl.ds(0, 16)] = ibuf[b, 1, pl.ds(0, 16)]
                scidx[b, pl.ds(16, 16)] = ibuf[b, 1, pl.ds(16, 16)]
                _compute_chunk(srows.at[b], drows.at[b], msg.at[b], aw_regs,
                               lane)
                pltpu.async_copy(msg.at[b], shared.at[scidx.at[b]], csems[b],
                                 add=True)

            @pl.when(_valid(3))
            def _pref():
                _issue_idx(jj + 3, b)
        return 0

    lax.fori_loop(0, (NITER + NB - 1) // NB, outer, 0)
    for b in range(NB):
        pltpu.make_async_copy(msg.at[b], shared.at[scidx.at[b]],
                              csems[b]).wait()
    plsc.subcore_barrier()

    @pl.when(s < INIT_W)
    def _drain():
        pltpu.sync_copy(shared.at[pl.ds(row0, ROWS_PT)],
                        out_hbm.at[c, pl.ds(row0, ROWS_PT)])


def kernel(h_int, edge_index, pair_W, pair_b, attn_w, value_W, out_W, out_b,
           phase_W, phase_b):
    tarr = jnp.array(_TARR, jnp.int32)
    w_src = jnp.concatenate([pair_W[0, :D], pair_W[1, :D], value_W[0],
                             value_W[1]], axis=1)
    w_dst = jnp.concatenate([pair_W[0, D:], pair_W[1, D:]], axis=1)
    w_val_t = jnp.concatenate([value_W[0], value_W[1]], axis=1)[:, tarr]
    b_dst = jnp.concatenate([pair_b[0], pair_b[1]])[None, :]
    aw = jnp.concatenate([attn_w[0], attn_w[1]])
    aw_p = aw[tarr]
    out_w_p = out_W[tarr, :]

    grid = (N // RB,)
    stab, dtab, init = pl.pallas_call(
        _proj_body,
        grid=grid,
        in_specs=[
            pl.BlockSpec((RB, D), lambda i: (i, 0)),
            pl.BlockSpec((D, 2 * D), lambda i: (0, 0)),
            pl.BlockSpec((D, D), lambda i: (0, 0)),
            pl.BlockSpec((D, D), lambda i: (0, 0)),
            pl.BlockSpec((1, D), lambda i: (0, 0)),
            pl.BlockSpec((1, D), lambda i: (0, 0)),
        ],
        out_specs=[
            pl.BlockSpec((RB, 2 * D), lambda i: (i, 0)),
            pl.BlockSpec((RB, D), lambda i: (i, 0)),
            pl.BlockSpec((RB, AC), lambda i: (i, 0)),
        ],
        out_shape=[
            jax.ShapeDtypeStruct((N, 2 * D), jnp.bfloat16),
            jax.ShapeDtypeStruct((N, D), jnp.bfloat16),
            jax.ShapeDtypeStruct((N, AC), jnp.float32),
        ],
    )(h_int, w_src, w_dst, w_val_t, b_dst, aw[None, :])

    edge_fn = pl.kernel(
        _edge_body,
        out_type=jax.ShapeDtypeStruct((NC, N, AC), jnp.float32),
        mesh=plsc.VectorSubcoreMesh(core_axis_name="c", subcore_axis_name="s"),
        scratch_types=[
            pltpu.VMEM((NB, 2, C), jnp.int32),
            pltpu.VMEM((NB, C), jnp.int32),
            pltpu.VMEM((NB, C, 2 * D), jnp.bfloat16),
            pltpu.VMEM((NB, C, D), jnp.bfloat16),
            pltpu.VMEM((NB, C, AC), jnp.float32),
            pltpu.VMEM((D,), jnp.float32),
            pltpu.VMEM_SHARED((N, AC), jnp.float32),
        ] + [pltpu.SemaphoreType.DMA] * 12,
        compiler_params=pltpu.CompilerParams(use_tc_tiling_on_sc=False,
                                             needs_layout_passes=False),
    )
    accs = edge_fn(edge_index, stab, dtab, aw_p, init)

    probs = pl.pallas_call(
        _final_body,
        grid=grid,
        in_specs=[
            pl.BlockSpec((NC, RB, AC), lambda i: (0, i, 0)),
            pl.BlockSpec((D, D), lambda i: (0, 0)),
            pl.BlockSpec((1, D), lambda i: (0, 0)),
            pl.BlockSpec((D, 4), lambda i: (0, 0)),
            pl.BlockSpec((1, 4), lambda i: (0, 0)),
        ],
        out_specs=pl.BlockSpec((RB, 4), lambda i: (i, 0)),
        out_shape=jax.ShapeDtypeStruct((N, 4), jnp.float32),
    )(accs, out_w_p, out_b[None, :], phase_W, phase_b[None, :])
    return probs


# speedup vs baseline: 1.4445x; 1.4445x over previous
import functools

import jax
import jax.numpy as jnp
from jax import lax
from jax.experimental import pallas as pl
from jax.experimental.pallas import tpu as pltpu
from jax.experimental.pallas import tpu_sc as plsc

N = 10000
E = 320000
D = 128
HD = 64
AC = 144
NC = 2
NS = 16
NW = NC * NS
C = 40
NCHUNK = E // C
CPT = NCHUNK // NW
INIT_W = 10
ROWS_PT = N // INIT_W
RB = 2000

_TARR = tuple(32 * m + 2 * i + r
              for m in range(4) for r in range(2) for i in range(16))


def _proj_body(h_ref, ws_ref, wd_ref, wvt_ref, bd_ref, aw_ref,
               stab_ref, dtab_ref, init_ref):
    h = h_ref[...]
    s = jnp.dot(h, ws_ref[...], preferred_element_type=jnp.float32)
    dt = jnp.dot(h, wd_ref[...], preferred_element_type=jnp.float32) + bd_ref[0]
    stab_ref[...] = s.astype(jnp.bfloat16)
    dtab_ref[...] = dt.astype(jnp.bfloat16)
    t = s[:, :D] + dt
    zlr = 0.6 * t + 0.4 * jnp.abs(t)
    aw = aw_ref[0]
    l0 = jnp.sum(zlr[:, :HD] * aw[:HD], axis=1)
    l1 = jnp.sum(zlr[:, HD:] * aw[HD:], axis=1)
    w0 = jnp.exp(l0)
    w1 = jnp.exp(l1)
    vq = jnp.dot(h, wvt_ref[...], preferred_element_type=jnp.float32)
    msg = jnp.concatenate(
        [w0[:, None] * vq[:, :HD],
         w1[:, None] * vq[:, HD:],
         w0[:, None], w1[:, None],
         jnp.zeros((s.shape[0], AC - D - 2), jnp.float32)], axis=1)
    init_ref[...] = 0.5 * msg


def _final_body(acc_ref, ow_ref, ob_ref, pw_ref, pb_ref, out_ref):
    a = acc_ref[0] + acc_ref[1]
    hc = jnp.concatenate(
        [a[:, :HD] / a[:, D:D + 1], a[:, HD:D] / a[:, D + 1:D + 2]], axis=1)
    h2 = jnp.maximum(
        jnp.dot(hc, ow_ref[...], preferred_element_type=jnp.float32) + ob_ref[0],
        0.0)
    lg = jnp.dot(h2, pw_ref[...], preferred_element_type=jnp.float32) + pb_ref[0]
    m = jnp.max(lg, axis=1, keepdims=True)
    e = jnp.exp(lg - m)
    out_ref[...] = e / jnp.sum(e, axis=1, keepdims=True)


def _compute_chunk(srows_b, drows_b, msg_b, aw_regs, lane):
    @plsc.parallel_loop(0, C, unroll=4)
    def eloop(e):
        ls = []
        for h in range(2):
            parts = []
            for j in (2 * h, 2 * h + 1):
                t = (srows_b[e, pl.ds(j * 32, 32)] +
                     drows_b[e, pl.ds(j * 32, 32)])
                ta, tb = plsc.unpack(t, format=plsc.PackFormat.INTERLEAVED,
                                     preferred_element_type=jnp.float32)
                for r, tt in enumerate((ta, tb)):
                    lr = 0.6 * tt + 0.4 * jnp.abs(tt)
                    parts.append(lr * aw_regs[2 * j + r])
            ls.append(jnp.sum((parts[0] + parts[1]) + (parts[2] + parts[3])))
        w0 = jnp.exp(jnp.full((16,), ls[0], jnp.float32))
        w1 = jnp.exp(jnp.full((16,), ls[1], jnp.float32))
        msg_b[e, pl.ds(D, 16)] = jnp.where(
            lane == 0, w0, jnp.where(lane == 1, w1, 0.0))
        for m in range(4):
            w = w0 if m < 2 else w1
            v = srows_b[e, pl.ds(D + m * 32, 32)]
            va, vb = plsc.unpack(v, format=plsc.PackFormat.INTERLEAVED,
                                 preferred_element_type=jnp.float32)
            msg_b[e, pl.ds(m * 32, 16)] = va * w
            msg_b[e, pl.ds(m * 32 + 16, 16)] = vb * w


def _edge_body(ei_hbm, stab_hbm, dtab_hbm, aw_hbm, init_hbm, out_hbm,
               ibuf, scidx, srows, drows, msg, aw_v, shared,
               isem0, isem1, ssem0, ssem1, dsem0, dsem1, csem0, csem1):
    c = lax.axis_index("c")
    s = lax.axis_index("s")
    wid = s * NC + c

    isems = (isem0, isem1)
    ssems = (ssem0, ssem1)
    dsems = (dsem0, dsem1)
    csems = (csem0, csem1)

    pltpu.sync_copy(aw_hbm, aw_v)
    aw_regs = [aw_v[pl.ds(j * 16, 16)] for j in range(8)]
    lane = lax.iota(jnp.int32, 16)

    row0 = s * ROWS_PT

    @pl.when(s < INIT_W)
    def _init():
        pltpu.sync_copy(init_hbm.at[pl.ds(row0, ROWS_PT)],
                        shared.at[pl.ds(row0, ROWS_PT)])

    plsc.subcore_barrier()

    def _issue_idx(j, b):
        return pltpu.async_copy(
            ei_hbm.at[:, pl.ds((j * NW + wid) * C, C)], ibuf.at[b], isems[b])

    def _issue_gathers(b):
        pltpu.async_copy(stab_hbm.at[ibuf.at[b, 0]], srows.at[b], ssems[b])
        pltpu.async_copy(dtab_hbm.at[ibuf.at[b, 1]], drows.at[b], dsems[b])

    def _wait_gathers(b):
        pltpu.make_async_copy(stab_hbm.at[ibuf.at[b, 0]], srows.at[b],
                              ssems[b]).wait()
        pltpu.make_async_copy(dtab_hbm.at[ibuf.at[b, 1]], drows.at[b],
                              dsems[b]).wait()

    _issue_idx(0, 0).wait()
    _issue_idx(1, 1)
    _issue_gathers(0)

    def outer(i, _):
        for b in range(2):
            jj = i * 2 + b
            nb = 1 - b
            _wait_gathers(b)

            @pl.when(jj + 1 < CPT)
            def _nxt():
                pltpu.make_async_copy(
                    ei_hbm.at[:, pl.ds(((jj + 1) * NW + wid) * C, C)],
                    ibuf.at[nb], isems[nb]).wait()
                _issue_gathers(nb)

            @pl.when(i >= 1)
            def _wsc():
                pltpu.make_async_copy(msg.at[b], shared.at[scidx.at[b]],
                                      csems[b]).wait()

            scidx[b, pl.ds(0, 16)] = ibuf[b, 1, pl.ds(0, 16)]
            scidx[b, pl.ds(16, 16)] = ibuf[b, 1, pl.ds(16, 16)]
            scidx[b, pl.ds(24, 16)] = ibuf[b, 1, pl.ds(24, 16)]

            _compute_chunk(srows.at[b], drows.at[b], msg.at[b], aw_regs, lane)
            pltpu.async_copy(msg.at[b], shared.at[scidx.at[b]], csems[b],
                             add=True)

            @pl.when(jj + 2 < CPT)
            def _pref():
                _issue_idx(jj + 2, b)
        return 0

    lax.fori_loop(0, CPT // 2, outer, 0)
    for b in range(2):
        pltpu.make_async_copy(msg.at[b], shared.at[scidx.at[b]],
                              csems[b]).wait()
    plsc.subcore_barrier()

    @pl.when(s < INIT_W)
    def _drain():
        pltpu.sync_copy(shared.at[pl.ds(row0, ROWS_PT)],
                        out_hbm.at[c, pl.ds(row0, ROWS_PT)])


def kernel(h_int, edge_index, pair_W, pair_b, attn_w, value_W, out_W, out_b,
           phase_W, phase_b):
    tarr = jnp.array(_TARR, jnp.int32)
    w_src = jnp.concatenate([pair_W[0, :D], pair_W[1, :D], value_W[0],
                             value_W[1]], axis=1)
    w_dst = jnp.concatenate([pair_W[0, D:], pair_W[1, D:]], axis=1)
    w_val_t = jnp.concatenate([value_W[0], value_W[1]], axis=1)[:, tarr]
    b_dst = jnp.concatenate([pair_b[0], pair_b[1]])[None, :]
    aw = jnp.concatenate([attn_w[0], attn_w[1]])
    aw_p = aw[tarr]
    out_w_p = out_W[tarr, :]

    grid = (N // RB,)
    stab, dtab, init = pl.pallas_call(
        _proj_body,
        grid=grid,
        in_specs=[
            pl.BlockSpec((RB, D), lambda i: (i, 0)),
            pl.BlockSpec((D, 2 * D), lambda i: (0, 0)),
            pl.BlockSpec((D, D), lambda i: (0, 0)),
            pl.BlockSpec((D, D), lambda i: (0, 0)),
            pl.BlockSpec((1, D), lambda i: (0, 0)),
            pl.BlockSpec((1, D), lambda i: (0, 0)),
        ],
        out_specs=[
            pl.BlockSpec((RB, 2 * D), lambda i: (i, 0)),
            pl.BlockSpec((RB, D), lambda i: (i, 0)),
            pl.BlockSpec((RB, AC), lambda i: (i, 0)),
        ],
        out_shape=[
            jax.ShapeDtypeStruct((N, 2 * D), jnp.bfloat16),
            jax.ShapeDtypeStruct((N, D), jnp.bfloat16),
            jax.ShapeDtypeStruct((N, AC), jnp.float32),
        ],
    )(h_int, w_src, w_dst, w_val_t, b_dst, aw[None, :])

    edge_fn = pl.kernel(
        _edge_body,
        out_type=jax.ShapeDtypeStruct((NC, N, AC), jnp.float32),
        mesh=plsc.VectorSubcoreMesh(core_axis_name="c", subcore_axis_name="s"),
        scratch_types=[
            pltpu.VMEM((2, 2, C), jnp.int32),
            pltpu.VMEM((2, C), jnp.int32),
            pltpu.VMEM((2, C, 2 * D), jnp.bfloat16),
            pltpu.VMEM((2, C, D), jnp.bfloat16),
            pltpu.VMEM((2, C, AC), jnp.float32),
            pltpu.VMEM((D,), jnp.float32),
            pltpu.VMEM_SHARED((N, AC), jnp.float32),
            pltpu.SemaphoreType.DMA,
            pltpu.SemaphoreType.DMA,
            pltpu.SemaphoreType.DMA,
            pltpu.SemaphoreType.DMA,
            pltpu.SemaphoreType.DMA,
            pltpu.SemaphoreType.DMA,
            pltpu.SemaphoreType.DMA,
            pltpu.SemaphoreType.DMA,
        ],
        compiler_params=pltpu.CompilerParams(use_tc_tiling_on_sc=False,
                                             needs_layout_passes=False),
    )
    accs = edge_fn(edge_index, stab, dtab, aw_p, init)

    probs = pl.pallas_call(
        _final_body,
        grid=grid,
        in_specs=[
            pl.BlockSpec((NC, RB, AC), lambda i: (0, i, 0)),
            pl.BlockSpec((D, D), lambda i: (0, 0)),
            pl.BlockSpec((1, D), lambda i: (0, 0)),
            pl.BlockSpec((D, 4), lambda i: (0, 0)),
            pl.BlockSpec((1, 4), lambda i: (0, 0)),
        ],
        out_specs=pl.BlockSpec((RB, 4), lambda i: (i, 0)),
        out_shape=jax.ShapeDtypeStruct((N, 4), jnp.float32),
    )(accs, out_w_p, out_b[None, :], phase_W, phase_b[None, :])
    return probs
